# Initial kernel scaffold; baseline (speedup 1.0000x reference)
import jax
import jax.numpy as jnp
from jax.experimental import pallas as pl


def kernel(indices, table):
    raise NotImplementedError("write your pallas kernel here")


# speedup vs baseline: 5.7618x; 5.7618x over previous
import jax
import jax.numpy as jnp
from jax.experimental import pallas as pl
from jax.experimental.pallas import tpu as pltpu

_BATCH_BLOCK = 1024


def _select_body(idx_ref, table_ref, out_ref):
    i = pl.program_id(0)
    idx = idx_ref[0, pl.ds(i * _BATCH_BLOCK, _BATCH_BLOCK)]
    idx2 = idx.reshape(_BATCH_BLOCK, 1)
    t0 = table_ref[0:1, :]
    t1 = table_ref[1:2, :]
    out_ref[...] = jnp.where(idx2 == 0, t0, t1)


def kernel(indices, table):
    batch = indices.shape[0]
    embed = table.shape[1]
    num_blocks = batch // _BATCH_BLOCK
    idx2d = indices.astype(jnp.int32).reshape(1, batch)
    return pl.pallas_call(
        _select_body,
        grid=(num_blocks,),
        in_specs=[
            pl.BlockSpec((1, batch), lambda i: (0, 0)),
            pl.BlockSpec((2, embed), lambda i: (0, 0)),
        ],
        out_specs=pl.BlockSpec((_BATCH_BLOCK, embed), lambda i: (i, 0)),
        out_shape=jax.ShapeDtypeStruct((batch, embed), table.dtype),
    )(idx2d, table)
